# baseline (speedup 1.0000x reference)
import functools

import jax
import jax.numpy as jnp
from jax import lax
from jax.experimental import pallas as pl
from jax.experimental.pallas import tpu as pltpu
from jax.experimental.pallas import tpu_sc as plsc

NC, NS, L = 2, 16, 16
H = 256
W = 256
C = 256
CV = C // L
WCHUNK = 32
NBUF = 3
SC_ROWS = 16
TCBLK = 8


def _tree_max(vals):
    vals = list(vals)
    while len(vals) > 1:
        nxt = [jnp.maximum(vals[i], vals[i + 1])
               for i in range(0, len(vals) - 1, 2)]
        if len(vals) % 2:
            nxt.append(vals[-1])
        vals = nxt
    return vals[0]


def _roi_pool_sc(fmap2d, bounds):
    mesh = plsc.VectorSubcoreMesh(
        core_axis_name="c", subcore_axis_name="s",
        num_cores=NC, num_subcores=NS)

    @functools.partial(
        pl.kernel,
        out_type=jax.ShapeDtypeStruct((8 * C,), jnp.float32),
        mesh=mesh,
        scratch_types=[
            pltpu.VMEM((L,), jnp.int32),
            pltpu.VMEM((NBUF, WCHUNK, C), jnp.float32),
            pltpu.VMEM((4, C), jnp.float32),
            pltpu.VMEM((NS, C), jnp.float32),
            pltpu.VMEM((C,), jnp.float32),
            pltpu.VMEM_SHARED((4, NS, C), jnp.float32),
        ] + [pltpu.SemaphoreType.DMA] * NBUF,
        compiler_params=pltpu.CompilerParams(needs_layout_passes=False),
    )
    def k(fmap_hbm, bounds_hbm, out_hbm, bvec, chunk, acc, red, orow, shared,
          *sems):
        core = lax.axis_index("c")
        sid = lax.axis_index("s")

        pltpu.sync_copy(bounds_hbm, bvec)
        b = bvec[...]
        lane = lax.iota(jnp.int32, L)
        neg = jnp.full((L,), jnp.int32(-(2**31)), dtype=jnp.int32)

        def sc(j):
            return lax.reduce_max(jnp.where(lane == j, b, neg), (0,))

        rois = []
        for i in range(2):
            rois.append(tuple(sc(i * 8 + j) for j in range(6)))

        def cw(a, c_):
            return jnp.where(core == 0, a, c_)

        def quad(i, r, s):
            h0, h1, h2, w0, w1, w2 = rois[i]
            ra = (h0, h1)[r]
            rb = jnp.minimum(ra + SC_ROWS, (h1, h2)[r])
            wa = (w0, w1)[s]
            wb = (w1, w2)[s]
            return ra, rb, wa, wb, i * 4 + r * 2 + s

        assign0 = [quad(0, 0, 0), quad(0, 1, 1), quad(1, 0, 1), quad(1, 1, 0)]
        assign1 = [quad(0, 0, 1), quad(0, 1, 0), quad(1, 0, 0), quad(1, 1, 1)]
        segs = [tuple(cw(a, c_) for a, c_ in zip(sa, sb))
                for sa, sb in zip(assign0, assign1)]

        ninf = jnp.full((L,), -jnp.inf, dtype=jnp.float32)
        for q in range(4):
            for kk in range(CV):
                acc[q, pl.ds(kk * L, L)] = ninf

        a0s, ras, was, wbs, ncws, cnts, outrows = [], [], [], [], [], [], []
        for (ra, rb, wa, wb, orow_id) in segs:
            a0 = (wa // 8) * 8
            nrows = (rb - ra - sid + NS - 1) // NS
            ncw = (wb - a0 + WCHUNK - 1) // WCHUNK
            a0s.append(a0)
            ras.append(ra)
            was.append(wa)
            wbs.append(wb)
            ncws.append(ncw)
            cnts.append(nrows * ncw)
            outrows.append(orow_id)
        cum1 = cnts[0]
        cum2 = cum1 + cnts[1]
        cum3 = cum2 + cnts[2]
        total = cum3 + cnts[3]

        def sel(v, vals):
            r = vals[3]
            r = jnp.where(v == 2, vals[2], r)
            r = jnp.where(v == 1, vals[1], r)
            return jnp.where(v == 0, vals[0], r)

        def chunk_params(idx):
            v = ((idx >= cum1).astype(jnp.int32)
                 + (idx >= cum2).astype(jnp.int32)
                 + (idx >= cum3).astype(jnp.int32))
            local = idx - sel(v, [0, cum1, cum2, cum3])
            ncw = sel(v, ncws)
            t = lax.div(local, ncw)
            u = local - t * ncw
            h = sel(v, ras) + sid + t * NS
            start = jnp.minimum(sel(v, a0s) + u * WCHUNK, W - WCHUNK)
            return v, h, start

        def chunk_src(idx):
            _v, h, start = chunk_params(idx)
            return fmap_hbm.at[pl.ds(h * W + start, WCHUNK)]

        for bslot in range(NBUF):
            @pl.when(bslot < total)
            def _(bslot=bslot):
                pltpu.async_copy(chunk_src(jnp.int32(bslot)),
                                 chunk.at[bslot], sems[bslot])

        dummy_src = fmap_hbm.at[pl.ds(0, WCHUNK)]
        ng = (total + NBUF - 1) // NBUF

        def group_body(g, _):
            base = g * NBUF
            for bslot in range(NBUF):
                idx = base + bslot

                @pl.when(idx < total)
                def _(idx=idx, bslot=bslot):
                    pltpu.make_async_copy(
                        dummy_src, chunk.at[bslot], sems[bslot]).wait()
                    v, _h, start = chunk_params(idx)
                    nl = sel(v, was) - start
                    nr = start + WCHUNK - sel(v, wbs)

                    def clear(j, _, bslot=bslot):
                        for kk in range(CV):
                            chunk[bslot, j, pl.ds(kk * L, L)] = ninf
                        return 0

                    @pl.when(nl > 0)
                    def _(bslot=bslot):
                        lax.fori_loop(0, nl, clear, 0)

                    @pl.when(nr > 0)
                    def _(bslot=bslot):
                        lax.fori_loop(WCHUNK - nr, WCHUNK, clear, 0)

                    for kk in range(CV):
                        m = _tree_max([
                            chunk[bslot, j, pl.ds(kk * L, L)]
                            for j in range(WCHUNK)
                        ])
                        acc[v, pl.ds(kk * L, L)] = jnp.maximum(
                            acc[v, pl.ds(kk * L, L)], m)

                    @pl.when(idx + NBUF < total)
                    def _(idx=idx, bslot=bslot):
                        pltpu.async_copy(chunk_src(idx + NBUF),
                                         chunk.at[bslot], sems[bslot])
            return 0

        lax.fori_loop(0, ng, group_body, 0)

        for q in range(4):
            pltpu.sync_copy(acc.at[q], shared.at[q, sid])
        plsc.subcore_barrier()

        @pl.when(sid < 4)
        def _():
            pltpu.sync_copy(shared.at[sid], red)
            qrow = sel(sid, outrows)
            for kk in range(CV):
                m = _tree_max([red[t, pl.ds(kk * L, L)] for t in range(NS)])
                orow[pl.ds(kk * L, L)] = m
            pltpu.sync_copy(orow, out_hbm.at[pl.ds(qrow * C, C)])

    return k(fmap2d, bounds)


def _tc_body(bounds_ref, x_ref, out_ref):
    i = pl.program_id(0)

    @pl.when(i == 0)
    def _():
        out_ref[...] = jnp.full((8, C), -jnp.inf, dtype=jnp.float32)

    ninf = jnp.float32(-jnp.inf)
    colvec = lax.broadcasted_iota(jnp.int32, (W, 1), 0)
    bnds = []
    for i_roi in range(2):
        bnds.append(tuple(bounds_ref[i_roi * 8 + j] for j in range(6)))
    cms = {}
    for i_roi in range(2):
        _h0, _h1, _h2, w0, w1, w2 = bnds[i_roi]
        for s in range(2):
            wa = (w0, w1)[s]
            wb = (w1, w2)[s]
            cms[(i_roi, s)] = (colvec >= wa) & (colvec < wb)

    rowred = {}
    for j in range(TCBLK):
        xr = x_ref[j]
        for i_roi in range(2):
            for s in range(2):
                rowred[(i_roi, s, j)] = jnp.max(
                    jnp.where(cms[(i_roi, s)], xr, ninf),
                    axis=0, keepdims=True)

    for i_roi in range(2):
        h0, h1, h2 = bnds[i_roi][:3]
        for s in range(2):
            for r in range(2):
                ra = (h0, h1)[r]
                rb = (h1, h2)[r]
                ra_eff = jnp.minimum(ra + SC_ROWS, rb)
                vals = []
                for j in range(TCBLK):
                    rscal = i * TCBLK + j
                    ok = (rscal >= ra_eff) & (rscal < rb)
                    vals.append(jnp.where(ok, rowred[(i_roi, s, j)], ninf))
                red = _tree_max(vals)
                q = i_roi * 4 + r * 2 + s
                out_ref[pl.ds(q, 1), :] = jnp.maximum(
                    out_ref[pl.ds(q, 1), :], red)


def _roi_pool_tc(fmap3d, bounds):
    return pl.pallas_call(
        _tc_body,
        grid=(H // TCBLK,),
        in_specs=[
            pl.BlockSpec(memory_space=pltpu.MemorySpace.SMEM),
            pl.BlockSpec((TCBLK, W, C), lambda i: (i, 0, 0)),
        ],
        out_specs=pl.BlockSpec((8, C), lambda i: (0, 0)),
        out_shape=jax.ShapeDtypeStruct((8, C), jnp.float32),
    )(bounds, fmap3d)


def kernel(conv_feature_map, roi_edges):
    n, h, w, c = conv_feature_map.shape
    e = roi_edges[:2]
    left = jnp.round(w * e[:, 0]).astype(jnp.int32)
    right = jnp.round(w * e[:, 1]).astype(jnp.int32)
    top = jnp.round(h * e[:, 2]).astype(jnp.int32)
    bottom = jnp.round(h * e[:, 3]).astype(jnp.int32)

    def mid(lo, hi):
        total = hi - lo
        xup = (total + 1) // 2
        m = jnp.where(xup >= total, xup - 1, xup)
        return lo + m

    h1 = mid(top, bottom)
    w1 = mid(left, right)
    zeros = jnp.zeros((2,), jnp.int32)
    bounds = jnp.stack(
        [top, h1, bottom, left, w1, right, zeros, zeros], axis=1
    ).reshape(16)

    res_sc = _roi_pool_sc(conv_feature_map.reshape(h * w, c), bounds)
    res_tc = _roi_pool_tc(conv_feature_map.reshape(h, w, c), bounds)
    out = jnp.maximum(res_sc.reshape(8, c), res_tc)
    return out.reshape(2, 1, 2, 2, c)


# speedup vs baseline: 1.1466x; 1.1466x over previous
import functools

import jax
import jax.numpy as jnp
from jax import lax
from jax.experimental import pallas as pl
from jax.experimental.pallas import tpu as pltpu
from jax.experimental.pallas import tpu_sc as plsc

NC, NS, L = 2, 16, 16
H = 256
W = 256
C = 256
CV = C // L
WCHUNK = 32
NBUF = 3
SC_ROWS = 8
TCBLK = 16


def _tree_max(vals):
    vals = list(vals)
    while len(vals) > 1:
        nxt = [jnp.maximum(vals[i], vals[i + 1])
               for i in range(0, len(vals) - 1, 2)]
        if len(vals) % 2:
            nxt.append(vals[-1])
        vals = nxt
    return vals[0]


def _roi_pool_sc(fmap2d, bounds):
    mesh = plsc.VectorSubcoreMesh(
        core_axis_name="c", subcore_axis_name="s",
        num_cores=NC, num_subcores=NS)

    @functools.partial(
        pl.kernel,
        out_type=jax.ShapeDtypeStruct((8 * C,), jnp.float32),
        mesh=mesh,
        scratch_types=[
            pltpu.VMEM((L,), jnp.int32),
            pltpu.VMEM((NBUF, WCHUNK, C), jnp.float32),
            pltpu.VMEM((4, C), jnp.float32),
            pltpu.VMEM((NS, C), jnp.float32),
            pltpu.VMEM((C,), jnp.float32),
            pltpu.VMEM_SHARED((4, NS, C), jnp.float32),
        ] + [pltpu.SemaphoreType.DMA] * NBUF,
        compiler_params=pltpu.CompilerParams(needs_layout_passes=False),
    )
    def k(fmap_hbm, bounds_hbm, out_hbm, bvec, chunk, acc, red, orow, shared,
          *sems):
        core = lax.axis_index("c")
        sid = lax.axis_index("s")

        pltpu.sync_copy(bounds_hbm, bvec)
        b = bvec[...]
        lane = lax.iota(jnp.int32, L)
        neg = jnp.full((L,), jnp.int32(-(2**31)), dtype=jnp.int32)

        def sc(j):
            return lax.reduce_max(jnp.where(lane == j, b, neg), (0,))

        rois = []
        for i in range(2):
            rois.append(tuple(sc(i * 8 + j) for j in range(6)))

        def cw(a, c_):
            return jnp.where(core == 0, a, c_)

        def quad(i, r, s):
            h0, h1, h2, w0, w1, w2 = rois[i]
            ra = (h0, h1)[r]
            rb = jnp.minimum(ra + SC_ROWS, (h1, h2)[r])
            wa = (w0, w1)[s]
            wb = (w1, w2)[s]
            return ra, rb, wa, wb, i * 4 + r * 2 + s

        assign0 = [quad(0, 0, 0), quad(0, 1, 1), quad(1, 0, 1), quad(1, 1, 0)]
        assign1 = [quad(0, 0, 1), quad(0, 1, 0), quad(1, 0, 0), quad(1, 1, 1)]
        segs = [tuple(cw(a, c_) for a, c_ in zip(sa, sb))
                for sa, sb in zip(assign0, assign1)]

        ninf = jnp.full((L,), -jnp.inf, dtype=jnp.float32)
        for q in range(4):
            for kk in range(CV):
                acc[q, pl.ds(kk * L, L)] = ninf

        a0s, ras, was, wbs, ncws, cnts, outrows = [], [], [], [], [], [], []
        for (ra, rb, wa, wb, orow_id) in segs:
            a0 = (wa // 8) * 8
            nrows = (rb - ra - sid + NS - 1) // NS
            ncw = (wb - a0 + WCHUNK - 1) // WCHUNK
            a0s.append(a0)
            ras.append(ra)
            was.append(wa)
            wbs.append(wb)
            ncws.append(ncw)
            cnts.append(nrows * ncw)
            outrows.append(orow_id)
        cum1 = cnts[0]
        cum2 = cum1 + cnts[1]
        cum3 = cum2 + cnts[2]
        total = cum3 + cnts[3]

        def sel(v, vals):
            r = vals[3]
            r = jnp.where(v == 2, vals[2], r)
            r = jnp.where(v == 1, vals[1], r)
            return jnp.where(v == 0, vals[0], r)

        def chunk_params(idx):
            v = ((idx >= cum1).astype(jnp.int32)
                 + (idx >= cum2).astype(jnp.int32)
                 + (idx >= cum3).astype(jnp.int32))
            local = idx - sel(v, [0, cum1, cum2, cum3])
            ncw = sel(v, ncws)
            t = lax.div(local, ncw)
            u = local - t * ncw
            h = sel(v, ras) + sid + t * NS
            start = jnp.minimum(sel(v, a0s) + u * WCHUNK, W - WCHUNK)
            return v, h, start

        def chunk_src(idx):
            _v, h, start = chunk_params(idx)
            return fmap_hbm.at[pl.ds(h * W + start, WCHUNK)]

        for bslot in range(NBUF):
            @pl.when(bslot < total)
            def _(bslot=bslot):
                pltpu.async_copy(chunk_src(jnp.int32(bslot)),
                                 chunk.at[bslot], sems[bslot])

        dummy_src = fmap_hbm.at[pl.ds(0, WCHUNK)]
        ng = (total + NBUF - 1) // NBUF

        def group_body(g, _):
            base = g * NBUF
            for bslot in range(NBUF):
                idx = base + bslot

                @pl.when(idx < total)
                def _(idx=idx, bslot=bslot):
                    pltpu.make_async_copy(
                        dummy_src, chunk.at[bslot], sems[bslot]).wait()
                    v, _h, start = chunk_params(idx)
                    nl = sel(v, was) - start
                    nr = start + WCHUNK - sel(v, wbs)

                    def clear(j, _, bslot=bslot):
                        for kk in range(CV):
                            chunk[bslot, j, pl.ds(kk * L, L)] = ninf
                        return 0

                    @pl.when(nl > 0)
                    def _(bslot=bslot):
                        lax.fori_loop(0, nl, clear, 0)

                    @pl.when(nr > 0)
                    def _(bslot=bslot):
                        lax.fori_loop(WCHUNK - nr, WCHUNK, clear, 0)

                    for kk in range(CV):
                        m = _tree_max([
                            chunk[bslot, j, pl.ds(kk * L, L)]
                            for j in range(WCHUNK)
                        ])
                        acc[v, pl.ds(kk * L, L)] = jnp.maximum(
                            acc[v, pl.ds(kk * L, L)], m)

                    @pl.when(idx + NBUF < total)
                    def _(idx=idx, bslot=bslot):
                        pltpu.async_copy(chunk_src(idx + NBUF),
                                         chunk.at[bslot], sems[bslot])
            return 0

        lax.fori_loop(0, ng, group_body, 0)

        for q in range(4):
            pltpu.sync_copy(acc.at[q], shared.at[q, sid])
        plsc.subcore_barrier()

        @pl.when(sid < 4)
        def _():
            pltpu.sync_copy(shared.at[sid], red)
            qrow = sel(sid, outrows)
            for kk in range(CV):
                m = _tree_max([red[t, pl.ds(kk * L, L)] for t in range(NS)])
                orow[pl.ds(kk * L, L)] = m
            pltpu.sync_copy(orow, out_hbm.at[pl.ds(qrow * C, C)])

    return k(fmap2d, bounds)


def _tc_body(bounds_ref, x_ref, out_ref):
    i = pl.program_id(0)

    @pl.when(i == 0)
    def _():
        out_ref[...] = jnp.full((8, C), -jnp.inf, dtype=jnp.float32)

    ninf = jnp.float32(-jnp.inf)
    colvec = lax.broadcasted_iota(jnp.int32, (W, 1), 0)
    bnds = []
    for i_roi in range(2):
        bnds.append(tuple(bounds_ref[i_roi * 8 + j] for j in range(6)))
    cms = {}
    for i_roi in range(2):
        _h0, _h1, _h2, w0, w1, w2 = bnds[i_roi]
        for s in range(2):
            wa = (w0, w1)[s]
            wb = (w1, w2)[s]
            cms[(i_roi, s)] = (colvec >= wa) & (colvec < wb)

    rowred = {}
    for j in range(TCBLK):
        xr = x_ref[j]
        for i_roi in range(2):
            for s in range(2):
                rowred[(i_roi, s, j)] = jnp.max(
                    jnp.where(cms[(i_roi, s)], xr, ninf),
                    axis=0, keepdims=True)

    for i_roi in range(2):
        h0, h1, h2 = bnds[i_roi][:3]
        for s in range(2):
            for r in range(2):
                ra = (h0, h1)[r]
                rb = (h1, h2)[r]
                ra_eff = jnp.minimum(ra + SC_ROWS, rb)
                vals = []
                for j in range(TCBLK):
                    rscal = i * TCBLK + j
                    ok = (rscal >= ra_eff) & (rscal < rb)
                    vals.append(jnp.where(ok, rowred[(i_roi, s, j)], ninf))
                red = _tree_max(vals)
                q = i_roi * 4 + r * 2 + s
                out_ref[pl.ds(q, 1), :] = jnp.maximum(
                    out_ref[pl.ds(q, 1), :], red)


def _roi_pool_tc(fmap3d, bounds):
    return pl.pallas_call(
        _tc_body,
        grid=(H // TCBLK,),
        in_specs=[
            pl.BlockSpec(memory_space=pltpu.MemorySpace.SMEM),
            pl.BlockSpec((TCBLK, W, C), lambda i: (i, 0, 0)),
        ],
        out_specs=pl.BlockSpec((8, C), lambda i: (0, 0)),
        out_shape=jax.ShapeDtypeStruct((8, C), jnp.float32),
    )(bounds, fmap3d)


def kernel(conv_feature_map, roi_edges):
    n, h, w, c = conv_feature_map.shape
    e = roi_edges[:2]
    left = jnp.round(w * e[:, 0]).astype(jnp.int32)
    right = jnp.round(w * e[:, 1]).astype(jnp.int32)
    top = jnp.round(h * e[:, 2]).astype(jnp.int32)
    bottom = jnp.round(h * e[:, 3]).astype(jnp.int32)

    def mid(lo, hi):
        total = hi - lo
        xup = (total + 1) // 2
        m = jnp.where(xup >= total, xup - 1, xup)
        return lo + m

    h1 = mid(top, bottom)
    w1 = mid(left, right)
    zeros = jnp.zeros((2,), jnp.int32)
    bounds = jnp.stack(
        [top, h1, bottom, left, w1, right, zeros, zeros], axis=1
    ).reshape(16)

    res_sc = _roi_pool_sc(conv_feature_map.reshape(h * w, c), bounds)
    res_tc = _roi_pool_tc(conv_feature_map.reshape(h, w, c), bounds)
    out = jnp.maximum(res_sc.reshape(8, c), res_tc)
    return out.reshape(2, 1, 2, 2, c)


# speedup vs baseline: 1.2015x; 1.0479x over previous
import functools

import jax
import jax.numpy as jnp
from jax import lax
from jax.experimental import pallas as pl
from jax.experimental.pallas import tpu as pltpu
from jax.experimental.pallas import tpu_sc as plsc

NC, NS, L = 2, 16, 16
H = 256
W = 256
C = 256
CV = C // L
WCHUNK = 32
NBUF = 3
SC_ROWS = 8
TCBLK = 32


def _tree_max(vals):
    vals = list(vals)
    while len(vals) > 1:
        nxt = [jnp.maximum(vals[i], vals[i + 1])
               for i in range(0, len(vals) - 1, 2)]
        if len(vals) % 2:
            nxt.append(vals[-1])
        vals = nxt
    return vals[0]


def _roi_pool_sc(fmap2d, bounds):
    mesh = plsc.VectorSubcoreMesh(
        core_axis_name="c", subcore_axis_name="s",
        num_cores=NC, num_subcores=NS)

    @functools.partial(
        pl.kernel,
        out_type=jax.ShapeDtypeStruct((8 * C,), jnp.float32),
        mesh=mesh,
        scratch_types=[
            pltpu.VMEM((L,), jnp.int32),
            pltpu.VMEM((NBUF, WCHUNK, C), jnp.float32),
            pltpu.VMEM((4, C), jnp.float32),
            pltpu.VMEM((NS, C), jnp.float32),
            pltpu.VMEM((C,), jnp.float32),
            pltpu.VMEM_SHARED((4, NS, C), jnp.float32),
        ] + [pltpu.SemaphoreType.DMA] * NBUF,
        compiler_params=pltpu.CompilerParams(needs_layout_passes=False),
    )
    def k(fmap_hbm, bounds_hbm, out_hbm, bvec, chunk, acc, red, orow, shared,
          *sems):
        core = lax.axis_index("c")
        sid = lax.axis_index("s")

        pltpu.sync_copy(bounds_hbm, bvec)
        b = bvec[...]
        lane = lax.iota(jnp.int32, L)
        neg = jnp.full((L,), jnp.int32(-(2**31)), dtype=jnp.int32)

        def sc(j):
            return lax.reduce_max(jnp.where(lane == j, b, neg), (0,))

        rois = []
        for i in range(2):
            rois.append(tuple(sc(i * 8 + j) for j in range(6)))

        def cw(a, c_):
            return jnp.where(core == 0, a, c_)

        def quad(i, r, s):
            h0, h1, h2, w0, w1, w2 = rois[i]
            ra = (h0, h1)[r]
            rb = jnp.minimum(ra + SC_ROWS, (h1, h2)[r])
            wa = (w0, w1)[s]
            wb = (w1, w2)[s]
            return ra, rb, wa, wb, i * 4 + r * 2 + s

        assign0 = [quad(0, 0, 0), quad(0, 1, 1), quad(1, 0, 1), quad(1, 1, 0)]
        assign1 = [quad(0, 0, 1), quad(0, 1, 0), quad(1, 0, 0), quad(1, 1, 1)]
        segs = [tuple(cw(a, c_) for a, c_ in zip(sa, sb))
                for sa, sb in zip(assign0, assign1)]

        ninf = jnp.full((L,), -jnp.inf, dtype=jnp.float32)
        for q in range(4):
            for kk in range(CV):
                acc[q, pl.ds(kk * L, L)] = ninf

        a0s, ras, was, wbs, ncws, cnts, outrows = [], [], [], [], [], [], []
        for (ra, rb, wa, wb, orow_id) in segs:
            a0 = (wa // 8) * 8
            nrows = (rb - ra - sid + NS - 1) // NS
            ncw = (wb - a0 + WCHUNK - 1) // WCHUNK
            a0s.append(a0)
            ras.append(ra)
            was.append(wa)
            wbs.append(wb)
            ncws.append(ncw)
            cnts.append(nrows * ncw)
            outrows.append(orow_id)
        cum1 = cnts[0]
        cum2 = cum1 + cnts[1]
        cum3 = cum2 + cnts[2]
        total = cum3 + cnts[3]

        def sel(v, vals):
            r = vals[3]
            r = jnp.where(v == 2, vals[2], r)
            r = jnp.where(v == 1, vals[1], r)
            return jnp.where(v == 0, vals[0], r)

        def chunk_params(idx):
            v = ((idx >= cum1).astype(jnp.int32)
                 + (idx >= cum2).astype(jnp.int32)
                 + (idx >= cum3).astype(jnp.int32))
            local = idx - sel(v, [0, cum1, cum2, cum3])
            ncw = sel(v, ncws)
            t = lax.div(local, ncw)
            u = local - t * ncw
            h = sel(v, ras) + sid + t * NS
            start = jnp.minimum(sel(v, a0s) + u * WCHUNK, W - WCHUNK)
            return v, h, start

        def chunk_src(idx):
            _v, h, start = chunk_params(idx)
            return fmap_hbm.at[pl.ds(h * W + start, WCHUNK)]

        for bslot in range(NBUF):
            @pl.when(bslot < total)
            def _(bslot=bslot):
                pltpu.async_copy(chunk_src(jnp.int32(bslot)),
                                 chunk.at[bslot], sems[bslot])

        dummy_src = fmap_hbm.at[pl.ds(0, WCHUNK)]
        ng = (total + NBUF - 1) // NBUF

        def group_body(g, _):
            base = g * NBUF
            for bslot in range(NBUF):
                idx = base + bslot

                @pl.when(idx < total)
                def _(idx=idx, bslot=bslot):
                    pltpu.make_async_copy(
                        dummy_src, chunk.at[bslot], sems[bslot]).wait()
                    v, _h, start = chunk_params(idx)
                    nl = sel(v, was) - start
                    nr = start + WCHUNK - sel(v, wbs)

                    def clear(j, _, bslot=bslot):
                        for kk in range(CV):
                            chunk[bslot, j, pl.ds(kk * L, L)] = ninf
                        return 0

                    @pl.when(nl > 0)
                    def _(bslot=bslot):
                        lax.fori_loop(0, nl, clear, 0)

                    @pl.when(nr > 0)
                    def _(bslot=bslot):
                        lax.fori_loop(WCHUNK - nr, WCHUNK, clear, 0)

                    for kk in range(CV):
                        m = _tree_max([
                            chunk[bslot, j, pl.ds(kk * L, L)]
                            for j in range(WCHUNK)
                        ])
                        acc[v, pl.ds(kk * L, L)] = jnp.maximum(
                            acc[v, pl.ds(kk * L, L)], m)

                    @pl.when(idx + NBUF < total)
                    def _(idx=idx, bslot=bslot):
                        pltpu.async_copy(chunk_src(idx + NBUF),
                                         chunk.at[bslot], sems[bslot])
            return 0

        lax.fori_loop(0, ng, group_body, 0)

        for q in range(4):
            pltpu.sync_copy(acc.at[q], shared.at[q, sid])
        plsc.subcore_barrier()

        @pl.when(sid < 4)
        def _():
            pltpu.sync_copy(shared.at[sid], red)
            qrow = sel(sid, outrows)
            for kk in range(CV):
                m = _tree_max([red[t, pl.ds(kk * L, L)] for t in range(NS)])
                orow[pl.ds(kk * L, L)] = m
            pltpu.sync_copy(orow, out_hbm.at[pl.ds(qrow * C, C)])

    return k(fmap2d, bounds)


def _tc_body(bounds_ref, x_ref, out_ref):
    i = pl.program_id(0)

    @pl.when(i == 0)
    def _():
        out_ref[...] = jnp.full((8, C), -jnp.inf, dtype=jnp.float32)

    ninf = jnp.float32(-jnp.inf)
    colvec = lax.broadcasted_iota(jnp.int32, (W, 1), 0)
    bnds = []
    for i_roi in range(2):
        bnds.append(tuple(bounds_ref[i_roi * 8 + j] for j in range(6)))
    cms = {}
    for i_roi in range(2):
        _h0, _h1, _h2, w0, w1, w2 = bnds[i_roi]
        for s in range(2):
            wa = (w0, w1)[s]
            wb = (w1, w2)[s]
            cms[(i_roi, s)] = (colvec >= wa) & (colvec < wb)

    rowred = {}
    for j in range(TCBLK):
        xr = x_ref[j]
        for i_roi in range(2):
            for s in range(2):
                rowred[(i_roi, s, j)] = jnp.max(
                    jnp.where(cms[(i_roi, s)], xr, ninf),
                    axis=0, keepdims=True)

    for i_roi in range(2):
        h0, h1, h2 = bnds[i_roi][:3]
        for s in range(2):
            for r in range(2):
                ra = (h0, h1)[r]
                rb = (h1, h2)[r]
                ra_eff = jnp.minimum(ra + SC_ROWS, rb)
                vals = []
                for j in range(TCBLK):
                    rscal = i * TCBLK + j
                    ok = (rscal >= ra_eff) & (rscal < rb)
                    vals.append(jnp.where(ok, rowred[(i_roi, s, j)], ninf))
                red = _tree_max(vals)
                q = i_roi * 4 + r * 2 + s
                out_ref[pl.ds(q, 1), :] = jnp.maximum(
                    out_ref[pl.ds(q, 1), :], red)


def _roi_pool_tc(fmap3d, bounds):
    return pl.pallas_call(
        _tc_body,
        grid=(H // TCBLK,),
        in_specs=[
            pl.BlockSpec(memory_space=pltpu.MemorySpace.SMEM),
            pl.BlockSpec((TCBLK, W, C), lambda i: (i, 0, 0)),
        ],
        out_specs=pl.BlockSpec((8, C), lambda i: (0, 0)),
        out_shape=jax.ShapeDtypeStruct((8, C), jnp.float32),
    )(bounds, fmap3d)


def kernel(conv_feature_map, roi_edges):
    n, h, w, c = conv_feature_map.shape
    e = roi_edges[:2]
    left = jnp.round(w * e[:, 0]).astype(jnp.int32)
    right = jnp.round(w * e[:, 1]).astype(jnp.int32)
    top = jnp.round(h * e[:, 2]).astype(jnp.int32)
    bottom = jnp.round(h * e[:, 3]).astype(jnp.int32)

    def mid(lo, hi):
        total = hi - lo
        xup = (total + 1) // 2
        m = jnp.where(xup >= total, xup - 1, xup)
        return lo + m

    h1 = mid(top, bottom)
    w1 = mid(left, right)
    zeros = jnp.zeros((2,), jnp.int32)
    bounds = jnp.stack(
        [top, h1, bottom, left, w1, right, zeros, zeros], axis=1
    ).reshape(16)

    res_sc = _roi_pool_sc(conv_feature_map.reshape(h * w, c), bounds)
    res_tc = _roi_pool_tc(conv_feature_map.reshape(h, w, c), bounds)
    out = jnp.maximum(res_sc.reshape(8, c), res_tc)
    return out.reshape(2, 1, 2, 2, c)
